# Initial kernel scaffold; baseline (speedup 1.0000x reference)
import functools

import jax
import jax.numpy as jnp
from jax import lax
from jax.experimental import pallas as pl
from jax.experimental.pallas import tpu as pltpu
from jax.experimental.pallas import tpu_sc as plsc

N = 10000
E = 320000
D = 128

NUM_TILES = 16
EDGES_PER_TILE = E // NUM_TILES
CHUNK = 80
NUM_CHUNKS = EDGES_PER_TILE // CHUNK
ROWS_PER_TILE = N // NUM_TILES

ROW_BLOCK = 2000


def _mm_body(x_self_ref, x_other_ref, wself_ref, wrel_ref, bias_ref,
             base_ref, table_ref):
    xs = x_self_ref[0]
    xo = x_other_ref[0]
    base_ref[0] = (
        jnp.dot(xs, wself_ref[0], preferred_element_type=jnp.float32)
        + bias_ref[0]
    )
    table_ref[0] = jnp.dot(xo, wrel_ref[0], preferred_element_type=jnp.float32)


def _tc_matmuls(x_cat, wself, wrel, bias):
    grid = (2, N // ROW_BLOCK)
    return pl.pallas_call(
        _mm_body,
        grid=grid,
        in_specs=[
            pl.BlockSpec((1, ROW_BLOCK, D), lambda c, r: (c, r, 0)),
            pl.BlockSpec((1, ROW_BLOCK, D), lambda c, r: (1 - c, r, 0)),
            pl.BlockSpec((1, D, D), lambda c, r: (c, 0, 0)),
            pl.BlockSpec((1, D, D), lambda c, r: (c, 0, 0)),
            pl.BlockSpec((1, 1, D), lambda c, r: (c, 0, 0)),
        ],
        out_specs=[
            pl.BlockSpec((1, ROW_BLOCK, D), lambda c, r: (c, r, 0)),
            pl.BlockSpec((1, ROW_BLOCK, D), lambda c, r: (c, r, 0)),
        ],
        out_shape=[
            jax.ShapeDtypeStruct((2, N, D), jnp.float32),
            jax.ShapeDtypeStruct((2, N, D), jnp.float32),
        ],
    )(x_cat, x_cat, wself, wrel, bias)


def _sc_body(table_hbm, base_hbm, src_hbm, dst_hbm, val_hbm, out_hbm,
             src_v, dst_v, val_v, rows_v, acc_sh, sem):
    c = lax.axis_index("c")
    s = lax.axis_index("s")
    row0 = s * ROWS_PER_TILE

    pltpu.sync_copy(base_hbm.at[c, pl.ds(row0, ROWS_PER_TILE)],
                    acc_sh.at[pl.ds(row0, ROWS_PER_TILE)])
    pltpu.sync_copy(src_hbm.at[c, s], src_v)
    pltpu.sync_copy(dst_hbm.at[c, s], dst_v)
    pltpu.sync_copy(val_hbm.at[c, s], val_v)
    plsc.subcore_barrier()

    def chunk_body(j, carry):
        pltpu.async_copy(table_hbm.at[src_v.at[j]], rows_v, sem).wait()

        def edge_body(e, carry2):
            v = val_v[j, e]
            for k in range(D // 16):
                sl = pl.ds(k * 16, 16)
                rows_v[e, sl] = rows_v[e, sl] * v
            return carry2

        lax.fori_loop(0, CHUNK, edge_body, 0)
        pltpu.sync_copy(rows_v, acc_sh.at[dst_v.at[j]], add=True)
        return carry

    lax.fori_loop(0, NUM_CHUNKS, chunk_body, 0)
    plsc.subcore_barrier()
    pltpu.sync_copy(acc_sh.at[pl.ds(row0, ROWS_PER_TILE)],
                    out_hbm.at[c, pl.ds(row0, ROWS_PER_TILE)])


_sc_spmm = functools.partial(
    pl.kernel,
    out_type=jax.ShapeDtypeStruct((2, N, D), jnp.float32),
    mesh=plsc.VectorSubcoreMesh(core_axis_name="c", subcore_axis_name="s"),
    scratch_types=[
        pltpu.VMEM((NUM_CHUNKS, CHUNK), jnp.int32),
        pltpu.VMEM((NUM_CHUNKS, CHUNK), jnp.int32),
        pltpu.VMEM((NUM_CHUNKS, CHUNK), jnp.float32),
        pltpu.VMEM((CHUNK, D), jnp.float32),
        pltpu.VMEM_SHARED((N, D), jnp.float32),
        pltpu.SemaphoreType.DMA,
    ],
)(_sc_body)


def kernel(x_a, x_b, adj_ab_indices, adj_ab_values, adj_ba_indices,
           adj_ba_values, W_rel_ab, w_self_a, bias_a, W_rel_ba, w_self_b,
           bias_b):
    x_cat = jnp.stack([x_a, x_b])
    wself = jnp.stack([w_self_a, w_self_b]) * 0.5
    wrel = jnp.stack([W_rel_ab, W_rel_ba]) * 0.5
    bias = jnp.stack([bias_a, bias_b])

    base, table = _tc_matmuls(x_cat, wself, wrel, bias)
    table_flat = table.reshape(2 * N, D)

    src = jnp.stack([
        adj_ab_indices[1].astype(jnp.int32),
        adj_ba_indices[1].astype(jnp.int32) + N,
    ]).reshape(2, NUM_TILES, NUM_CHUNKS, CHUNK)
    dst = jnp.stack([
        adj_ab_indices[0].astype(jnp.int32),
        adj_ba_indices[0].astype(jnp.int32),
    ]).reshape(2, NUM_TILES, NUM_CHUNKS, CHUNK)
    val = jnp.stack([adj_ab_values, adj_ba_values]).reshape(
        2, NUM_TILES, NUM_CHUNKS, CHUNK)

    out = _sc_spmm(table_flat, base, src, dst, val)
    return (out[0], out[1])


# speedup vs baseline: 3.0540x; 3.0540x over previous
import functools

import jax
import jax.numpy as jnp
from jax import lax
from jax.experimental import pallas as pl
from jax.experimental.pallas import tpu as pltpu
from jax.experimental.pallas import tpu_sc as plsc

N = 10000
E = 320000
D = 128

NUM_TILES = 16
CHUNK = 128
NUM_CHUNKS = 160
EDGES_PER_TILE = CHUNK * NUM_CHUNKS
E_PAD = EDGES_PER_TILE * NUM_TILES
NB = 16
NUM_BLOCKS = NUM_CHUNKS // NB
ROWS_PER_TILE = 624
TAIL_ROW0 = ROWS_PER_TILE * NUM_TILES
TAIL_ROWS = N - TAIL_ROW0

ROW_BLOCK = 2000


def _mm_body(x_self_ref, x_other_ref, wself_ref, wrel_ref, bias_ref,
             base_ref, table_ref):
    xs = x_self_ref[0]
    xo = x_other_ref[0]
    base_ref[0] = (
        jnp.dot(xs, wself_ref[0], preferred_element_type=jnp.float32)
        + bias_ref[0]
    )
    table_ref[0] = jnp.dot(xo, wrel_ref[0], preferred_element_type=jnp.float32)


def _tc_matmuls(x_cat, wself, wrel, bias):
    grid = (2, N // ROW_BLOCK)
    return pl.pallas_call(
        _mm_body,
        grid=grid,
        in_specs=[
            pl.BlockSpec((1, ROW_BLOCK, D), lambda c, r: (c, r, 0)),
            pl.BlockSpec((1, ROW_BLOCK, D), lambda c, r: (1 - c, r, 0)),
            pl.BlockSpec((1, D, D), lambda c, r: (c, 0, 0)),
            pl.BlockSpec((1, D, D), lambda c, r: (c, 0, 0)),
            pl.BlockSpec((1, 1, D), lambda c, r: (c, 0, 0)),
        ],
        out_specs=[
            pl.BlockSpec((1, ROW_BLOCK, D), lambda c, r: (c, r, 0)),
            pl.BlockSpec((1, ROW_BLOCK, D), lambda c, r: (c, r, 0)),
        ],
        out_shape=[
            jax.ShapeDtypeStruct((2, N, D), jnp.float32),
            jax.ShapeDtypeStruct((2, N, D), jnp.float32),
        ],
    )(x_cat, x_cat, wself, wrel, bias)


def _sc_body(table_hbm, base_hbm, src_hbm, dst_hbm, val_hbm, out_hbm,
             src_v, dst_v, val_v, rows_v, acc_sh, sem):
    c = lax.axis_index("c")
    s = lax.axis_index("s")
    row0 = s * ROWS_PER_TILE

    pltpu.sync_copy(base_hbm.at[c, pl.ds(row0, ROWS_PER_TILE)],
                    acc_sh.at[pl.ds(row0, ROWS_PER_TILE)])

    @pl.when(s == NUM_TILES - 1)
    def _init_tail():
        pltpu.sync_copy(base_hbm.at[c, pl.ds(TAIL_ROW0, TAIL_ROWS)],
                        acc_sh.at[pl.ds(TAIL_ROW0, TAIL_ROWS)])
    plsc.subcore_barrier()

    def block_body(b, carry):
        pltpu.sync_copy(src_hbm.at[c, s, pl.ds(b * NB, NB)], src_v)
        pltpu.sync_copy(dst_hbm.at[c, s, pl.ds(b * NB, NB)], dst_v)
        pltpu.sync_copy(val_hbm.at[c, s, pl.ds(b * NB, NB)], val_v)

        def chunk_body(j, carry2):
            pltpu.async_copy(table_hbm.at[src_v.at[j]], rows_v, sem).wait()

            def group_body(g, carry3):
                vv = val_v[j, pl.ds(g * 16, 16)]
                for i in range(16):
                    e = g * 16 + i
                    v = vv[i]
                    for k in range(D // 16):
                        sl = pl.ds(k * 16, 16)
                        rows_v[e, sl] = rows_v[e, sl] * v
                return carry3

            lax.fori_loop(0, CHUNK // 16, group_body, 0)
            pltpu.sync_copy(rows_v, acc_sh.at[dst_v.at[j]], add=True)
            return carry2

        lax.fori_loop(0, NB, chunk_body, 0)
        return carry

    lax.fori_loop(0, NUM_BLOCKS, block_body, 0)
    plsc.subcore_barrier()
    pltpu.sync_copy(acc_sh.at[pl.ds(row0, ROWS_PER_TILE)],
                    out_hbm.at[c, pl.ds(row0, ROWS_PER_TILE)])

    @pl.when(s == NUM_TILES - 1)
    def _write_tail():
        pltpu.sync_copy(acc_sh.at[pl.ds(TAIL_ROW0, TAIL_ROWS)],
                        out_hbm.at[c, pl.ds(TAIL_ROW0, TAIL_ROWS)])


_sc_spmm = functools.partial(
    pl.kernel,
    out_type=jax.ShapeDtypeStruct((2, N, D), jnp.float32),
    mesh=plsc.VectorSubcoreMesh(core_axis_name="c", subcore_axis_name="s"),
    scratch_types=[
        pltpu.VMEM((NB, CHUNK), jnp.int32),
        pltpu.VMEM((NB, CHUNK), jnp.int32),
        pltpu.VMEM((NB, CHUNK), jnp.float32),
        pltpu.VMEM((CHUNK, D), jnp.float32),
        pltpu.VMEM_SHARED((N, D), jnp.float32),
        pltpu.SemaphoreType.DMA,
    ],
)(_sc_body)


def kernel(x_a, x_b, adj_ab_indices, adj_ab_values, adj_ba_indices,
           adj_ba_values, W_rel_ab, w_self_a, bias_a, W_rel_ba, w_self_b,
           bias_b):
    x_cat = jnp.stack([x_a, x_b])
    wself = jnp.stack([w_self_a, w_self_b]) * 0.5
    wrel = jnp.stack([W_rel_ab, W_rel_ba]) * 0.5
    bias = jnp.stack([bias_a, bias_b])

    base, table = _tc_matmuls(x_cat, wself, wrel, bias)
    table_flat = table.reshape(2 * N, D)

    pad = E_PAD - E
    zpad_i = jnp.zeros((pad,), jnp.int32)
    zpad_f = jnp.zeros((pad,), jnp.float32)
    src = jnp.stack([
        jnp.concatenate([adj_ab_indices[1].astype(jnp.int32), zpad_i]),
        jnp.concatenate([adj_ba_indices[1].astype(jnp.int32) + N, zpad_i]),
    ]).reshape(2, NUM_TILES, NUM_CHUNKS, CHUNK)
    dst = jnp.stack([
        jnp.concatenate([adj_ab_indices[0].astype(jnp.int32), zpad_i]),
        jnp.concatenate([adj_ba_indices[0].astype(jnp.int32), zpad_i]),
    ]).reshape(2, NUM_TILES, NUM_CHUNKS, CHUNK)
    val = jnp.stack([
        jnp.concatenate([adj_ab_values, zpad_f]),
        jnp.concatenate([adj_ba_values, zpad_f]),
    ]).reshape(2, NUM_TILES, NUM_CHUNKS, CHUNK)

    out = _sc_spmm(table_flat, base, src, dst, val)
    return (out[0], out[1])


# speedup vs baseline: 3.4416x; 1.1269x over previous
import functools

import jax
import jax.numpy as jnp
from jax import lax
from jax.experimental import pallas as pl
from jax.experimental.pallas import tpu as pltpu
from jax.experimental.pallas import tpu_sc as plsc

N = 10000
E = 320000
D = 128

NUM_TILES = 16
CHUNK = 128
NUM_CHUNKS = 160
EDGES_PER_TILE = CHUNK * NUM_CHUNKS
E_PAD = EDGES_PER_TILE * NUM_TILES
NB = 16
NUM_BLOCKS = NUM_CHUNKS // NB
ROWS_PER_TILE = 624
TAIL_ROW0 = ROWS_PER_TILE * NUM_TILES
TAIL_ROWS = N - TAIL_ROW0

ROW_BLOCK = 2000


def _mm_body(x_self_ref, x_other_ref, wself_ref, wrel_ref, bias_ref,
             base_ref, table_ref):
    xs = x_self_ref[0]
    xo = x_other_ref[0]
    base_ref[0] = (
        jnp.dot(xs, wself_ref[0], preferred_element_type=jnp.float32)
        + bias_ref[0]
    )
    table_ref[0] = jnp.dot(xo, wrel_ref[0], preferred_element_type=jnp.float32)


def _tc_matmuls(x_cat, wself, wrel, bias):
    grid = (2, N // ROW_BLOCK)
    return pl.pallas_call(
        _mm_body,
        grid=grid,
        in_specs=[
            pl.BlockSpec((1, ROW_BLOCK, D), lambda c, r: (c, r, 0)),
            pl.BlockSpec((1, ROW_BLOCK, D), lambda c, r: (1 - c, r, 0)),
            pl.BlockSpec((1, D, D), lambda c, r: (c, 0, 0)),
            pl.BlockSpec((1, D, D), lambda c, r: (c, 0, 0)),
            pl.BlockSpec((1, 1, D), lambda c, r: (c, 0, 0)),
        ],
        out_specs=[
            pl.BlockSpec((1, ROW_BLOCK, D), lambda c, r: (c, r, 0)),
            pl.BlockSpec((1, ROW_BLOCK, D), lambda c, r: (c, r, 0)),
        ],
        out_shape=[
            jax.ShapeDtypeStruct((2, N, D), jnp.float32),
            jax.ShapeDtypeStruct((2, N, D), jnp.float32),
        ],
    )(x_cat, x_cat, wself, wrel, bias)


def _sc_body(table_hbm, base_hbm, src_hbm, dst_hbm, val_hbm, out_hbm,
             src_v, dst_v, val_v, rows_a, rows_b, acc_sh, sem_a, sem_b):
    c = lax.axis_index("c")
    s = lax.axis_index("s")
    row0 = s * ROWS_PER_TILE

    pltpu.sync_copy(base_hbm.at[c, pl.ds(row0, ROWS_PER_TILE)],
                    acc_sh.at[pl.ds(row0, ROWS_PER_TILE)])

    @pl.when(s == NUM_TILES - 1)
    def _init_tail():
        pltpu.sync_copy(base_hbm.at[c, pl.ds(TAIL_ROW0, TAIL_ROWS)],
                        acc_sh.at[pl.ds(TAIL_ROW0, TAIL_ROWS)])
    plsc.subcore_barrier()

    def scale_chunk(k, rows_ref):
        def group_body(g, carry):
            vv = val_v[k, pl.ds(g * 16, 16)]
            for i in range(16):
                e = g * 16 + i
                v = vv[i]
                for q in range(D // 16):
                    sl = pl.ds(q * 16, 16)
                    rows_ref[e, sl] = rows_ref[e, sl] * v
            return carry

        lax.fori_loop(0, CHUNK // 16, group_body, 0)

    def block_body(b, carry):
        pltpu.sync_copy(src_hbm.at[c, s, pl.ds(b * NB, NB)], src_v)
        pltpu.sync_copy(dst_hbm.at[c, s, pl.ds(b * NB, NB)], dst_v)
        pltpu.sync_copy(val_hbm.at[c, s, pl.ds(b * NB, NB)], val_v)
        pltpu.async_copy(table_hbm.at[src_v.at[0]], rows_a, sem_a)

        def pair_body(q, carry2):
            k0 = 2 * q
            pltpu.make_async_copy(table_hbm.at[src_v.at[k0]], rows_a,
                                  sem_a).wait()
            pltpu.async_copy(table_hbm.at[src_v.at[k0 + 1]], rows_b, sem_b)
            scale_chunk(k0, rows_a)
            pltpu.sync_copy(rows_a, acc_sh.at[dst_v.at[k0]], add=True)
            pltpu.make_async_copy(table_hbm.at[src_v.at[k0 + 1]], rows_b,
                                  sem_b).wait()

            @pl.when(k0 + 2 < NB)
            def _prefetch_next():
                pltpu.async_copy(table_hbm.at[src_v.at[k0 + 2]], rows_a,
                                 sem_a)

            scale_chunk(k0 + 1, rows_b)
            pltpu.sync_copy(rows_b, acc_sh.at[dst_v.at[k0 + 1]], add=True)
            return carry2

        lax.fori_loop(0, NB // 2, pair_body, 0)
        return carry

    lax.fori_loop(0, NUM_BLOCKS, block_body, 0)
    plsc.subcore_barrier()
    pltpu.sync_copy(acc_sh.at[pl.ds(row0, ROWS_PER_TILE)],
                    out_hbm.at[c, pl.ds(row0, ROWS_PER_TILE)])

    @pl.when(s == NUM_TILES - 1)
    def _write_tail():
        pltpu.sync_copy(acc_sh.at[pl.ds(TAIL_ROW0, TAIL_ROWS)],
                        out_hbm.at[c, pl.ds(TAIL_ROW0, TAIL_ROWS)])


_sc_spmm = functools.partial(
    pl.kernel,
    out_type=jax.ShapeDtypeStruct((2, N, D), jnp.float32),
    mesh=plsc.VectorSubcoreMesh(core_axis_name="c", subcore_axis_name="s"),
    scratch_types=[
        pltpu.VMEM((NB, CHUNK), jnp.int32),
        pltpu.VMEM((NB, CHUNK), jnp.int32),
        pltpu.VMEM((NB, CHUNK), jnp.float32),
        pltpu.VMEM((CHUNK, D), jnp.float32),
        pltpu.VMEM((CHUNK, D), jnp.float32),
        pltpu.VMEM_SHARED((N, D), jnp.float32),
        pltpu.SemaphoreType.DMA,
        pltpu.SemaphoreType.DMA,
    ],
)(_sc_body)


def kernel(x_a, x_b, adj_ab_indices, adj_ab_values, adj_ba_indices,
           adj_ba_values, W_rel_ab, w_self_a, bias_a, W_rel_ba, w_self_b,
           bias_b):
    x_cat = jnp.stack([x_a, x_b])
    wself = jnp.stack([w_self_a, w_self_b]) * 0.5
    wrel = jnp.stack([W_rel_ab, W_rel_ba]) * 0.5
    bias = jnp.stack([bias_a, bias_b])

    base, table = _tc_matmuls(x_cat, wself, wrel, bias)
    table_flat = table.reshape(2 * N, D)

    pad = E_PAD - E
    zpad_i = jnp.zeros((pad,), jnp.int32)
    zpad_f = jnp.zeros((pad,), jnp.float32)
    src = jnp.stack([
        jnp.concatenate([adj_ab_indices[1].astype(jnp.int32), zpad_i]),
        jnp.concatenate([adj_ba_indices[1].astype(jnp.int32) + N, zpad_i]),
    ]).reshape(2, NUM_TILES, NUM_CHUNKS, CHUNK)
    dst = jnp.stack([
        jnp.concatenate([adj_ab_indices[0].astype(jnp.int32), zpad_i]),
        jnp.concatenate([adj_ba_indices[0].astype(jnp.int32), zpad_i]),
    ]).reshape(2, NUM_TILES, NUM_CHUNKS, CHUNK)
    val = jnp.stack([
        jnp.concatenate([adj_ab_values, zpad_f]),
        jnp.concatenate([adj_ba_values, zpad_f]),
    ]).reshape(2, NUM_TILES, NUM_CHUNKS, CHUNK)

    out = _sc_spmm(table_flat, base, src, dst, val)
    return (out[0], out[1])
